# baseline (speedup 1.0000x reference)
import functools

import jax
import jax.numpy as jnp
from jax.experimental import pallas as pl

NPOINT = 1024
NSAMPLE = 32


def _fps_single(xyz):
    N = xyz.shape[0]

    def body(i, carry):
        dists, farthest, idxs = carry
        idxs = idxs.at[i].set(farthest)
        centroid = xyz[farthest]
        d = jnp.sum((xyz - centroid) ** 2, axis=-1)
        dists = jnp.minimum(dists, d)
        farthest = jnp.argmax(dists).astype(jnp.int32)
        return (dists, farthest, idxs)

    dists0 = jnp.full((N,), jnp.inf, dtype=xyz.dtype)
    idxs0 = jnp.zeros((NPOINT,), dtype=jnp.int32)
    _, _, idxs = jax.lax.fori_loop(0, NPOINT, body, (dists0, jnp.int32(0), idxs0))
    return idxs


def _index_points(points, idx):
    B = points.shape[0]
    batch_idx = jnp.arange(B).reshape((B,) + (1,) * (idx.ndim - 1))
    return points[batch_idx, idx]


def _square_distance(src, dst):
    dist = -2.0 * jnp.matmul(src, jnp.transpose(dst, (0, 2, 1)))
    dist = dist + jnp.sum(src ** 2, axis=-1)[:, :, None]
    dist = dist + jnp.sum(dst ** 2, axis=-1)[:, None, :]
    return dist


def _conv1x1(x, w, b):
    return jnp.einsum('oc,bcns->bons', w, x) + b[None, :, None, None]


def _batchnorm(x, g, beta, eps=1e-5):
    mean = jnp.mean(x, axis=(0, 2, 3), keepdims=True)
    var = jnp.var(x, axis=(0, 2, 3), keepdims=True)
    xhat = (x - mean) / jnp.sqrt(var + eps)
    return g[None, :, None, None] * xhat + beta[None, :, None, None]


def _swish(x):
    return x * jax.nn.sigmoid(x)


def _res_block(x, w, b, g, beta, sw, sb, sg, sbeta):
    residual = _batchnorm(_conv1x1(x, sw, sb), sg, sbeta)
    out = _swish(_batchnorm(_conv1x1(x, w, b), g, beta))
    return out + residual


def _max_pool_kernel(x_ref, o_ref):
    o_ref[...] = jnp.max(x_ref[...], axis=2)


def _max_over_samples(x):
    B, C, S, N = x.shape
    NCHUNK = 8
    return pl.pallas_call(
        _max_pool_kernel,
        grid=(B, NCHUNK),
        in_specs=[pl.BlockSpec((1, C, S, N // NCHUNK),
                               lambda b, n: (b, 0, 0, n))],
        out_specs=pl.BlockSpec((1, C, N // NCHUNK), lambda b, n: (b, 0, n)),
        out_shape=jax.ShapeDtypeStruct((B, C, N), x.dtype),
    )(x)


def kernel(xyz, points, b1_w, b1_b, b1_g, b1_beta, b1_sw, b1_sb, b1_sg, b1_sbeta, b2_w, b2_b, b2_g, b2_beta, b2_sw, b2_sb, b2_sg, b2_sbeta):
    fps_idx = jax.vmap(_fps_single)(xyz)
    new_xyz = _index_points(xyz, fps_idx)
    sqrdists = _square_distance(new_xyz, xyz)
    _, idx = jax.lax.top_k(-sqrdists, NSAMPLE)
    grouped_xyz = _index_points(xyz, idx) - new_xyz[:, :, None, :]
    grouped_points = _index_points(points, idx)
    grouped_points = jnp.concatenate([grouped_xyz, grouped_points], axis=-1)
    x = jnp.transpose(grouped_points, (0, 3, 2, 1))
    x = _res_block(x, b1_w, b1_b, b1_g, b1_beta, b1_sw, b1_sb, b1_sg, b1_sbeta)
    x = _res_block(x, b2_w, b2_b, b2_g, b2_beta, b2_sw, b2_sb, b2_sg, b2_sbeta)
    new_points = _max_over_samples(x)
    return (new_xyz, new_points)


# speedup vs baseline: 1.5945x; 1.5945x over previous
import functools

import jax
import jax.numpy as jnp
from jax.experimental import pallas as pl

NPOINT = 1024
NSAMPLE = 32


def _fps_single(xyz):
    N = xyz.shape[0]

    def body(i, carry):
        dists, farthest, idxs = carry
        idxs = idxs.at[i].set(farthest)
        centroid = xyz[farthest]
        d = jnp.sum((xyz - centroid) ** 2, axis=-1)
        dists = jnp.minimum(dists, d)
        farthest = jnp.argmax(dists).astype(jnp.int32)
        return (dists, farthest, idxs)

    dists0 = jnp.full((N,), jnp.inf, dtype=xyz.dtype)
    idxs0 = jnp.zeros((NPOINT,), dtype=jnp.int32)
    _, _, idxs = jax.lax.fori_loop(0, NPOINT, body, (dists0, jnp.int32(0), idxs0))
    return idxs


def _index_points(points, idx):
    B = points.shape[0]
    batch_idx = jnp.arange(B).reshape((B,) + (1,) * (idx.ndim - 1))
    return points[batch_idx, idx]


def _square_distance(src, dst):
    dist = -2.0 * jnp.matmul(src, jnp.transpose(dst, (0, 2, 1)))
    dist = dist + jnp.sum(src ** 2, axis=-1)[:, :, None]
    dist = dist + jnp.sum(dst ** 2, axis=-1)[:, None, :]
    return dist


def _conv1x1(x, w, b):
    return jnp.einsum('oc,bcns->bons', w, x) + b[None, :, None, None]


def _batchnorm(x, g, beta, eps=1e-5):
    mean = jnp.mean(x, axis=(0, 2, 3), keepdims=True)
    var = jnp.var(x, axis=(0, 2, 3), keepdims=True)
    xhat = (x - mean) / jnp.sqrt(var + eps)
    return g[None, :, None, None] * xhat + beta[None, :, None, None]


def _swish(x):
    return x * jax.nn.sigmoid(x)


def _res_block(x, w, b, g, beta, sw, sb, sg, sbeta):
    residual = _batchnorm(_conv1x1(x, sw, sb), sg, sbeta)
    out = _swish(_batchnorm(_conv1x1(x, w, b), g, beta))
    return out + residual


def _fps_pallas_kernel(p_ref, c_ref):
    _, B, N = p_ref.shape
    x = p_ref[0]
    y = p_ref[1]
    z = p_ref[2]
    iota = jax.lax.broadcasted_iota(jnp.int32, (B, N), 1)
    iota_p = jax.lax.broadcasted_iota(jnp.int32, (B, NPOINT), 1)

    def body(i, carry):
        dists, far, cx, cy, cz = carry
        onehot = iota == far
        fx = jnp.sum(jnp.where(onehot, x, 0.0), axis=1, keepdims=True)
        fy = jnp.sum(jnp.where(onehot, y, 0.0), axis=1, keepdims=True)
        fz = jnp.sum(jnp.where(onehot, z, 0.0), axis=1, keepdims=True)
        slot = iota_p == i
        cx = jnp.where(slot, fx, cx)
        cy = jnp.where(slot, fy, cy)
        cz = jnp.where(slot, fz, cz)
        d = (x - fx) ** 2 + (y - fy) ** 2 + (z - fz) ** 2
        dists = jnp.minimum(dists, d)
        m = jnp.max(dists, axis=1, keepdims=True)
        far = jnp.min(jnp.where(dists == m, iota, N), axis=1, keepdims=True)
        return dists, far, cx, cy, cz

    dists0 = jnp.full((B, N), jnp.inf, dtype=jnp.float32)
    far0 = jnp.zeros((B, 1), dtype=jnp.int32)
    c0 = jnp.zeros((B, NPOINT), dtype=jnp.float32)
    _, _, cx, cy, cz = jax.lax.fori_loop(
        0, NPOINT, body, (dists0, far0, c0, c0, c0))
    c_ref[0] = cx
    c_ref[1] = cy
    c_ref[2] = cz


def _fps_new_xyz(xyz, interpret=False):
    B, N, _ = xyz.shape
    planes = jnp.transpose(xyz, (2, 0, 1))
    c = pl.pallas_call(
        _fps_pallas_kernel,
        out_shape=jax.ShapeDtypeStruct((3, B, NPOINT), jnp.float32),
        interpret=interpret,
    )(planes)
    return jnp.transpose(c, (1, 2, 0))


def _max_pool_kernel(x_ref, o_ref):
    o_ref[...] = jnp.max(x_ref[...], axis=2)


def _max_over_samples(x):
    B, C, S, N = x.shape
    NCHUNK = 8
    return pl.pallas_call(
        _max_pool_kernel,
        grid=(B, NCHUNK),
        in_specs=[pl.BlockSpec((1, C, S, N // NCHUNK),
                               lambda b, n: (b, 0, 0, n))],
        out_specs=pl.BlockSpec((1, C, N // NCHUNK), lambda b, n: (b, 0, n)),
        out_shape=jax.ShapeDtypeStruct((B, C, N), x.dtype),
    )(x)


def kernel(xyz, points, b1_w, b1_b, b1_g, b1_beta, b1_sw, b1_sb, b1_sg, b1_sbeta, b2_w, b2_b, b2_g, b2_beta, b2_sw, b2_sb, b2_sg, b2_sbeta):
    new_xyz = _fps_new_xyz(xyz)
    sqrdists = _square_distance(new_xyz, xyz)
    _, idx = jax.lax.top_k(-sqrdists, NSAMPLE)
    grouped_xyz = _index_points(xyz, idx) - new_xyz[:, :, None, :]
    grouped_points = _index_points(points, idx)
    grouped_points = jnp.concatenate([grouped_xyz, grouped_points], axis=-1)
    x = jnp.transpose(grouped_points, (0, 3, 2, 1))
    x = _res_block(x, b1_w, b1_b, b1_g, b1_beta, b1_sw, b1_sb, b1_sg, b1_sbeta)
    x = _res_block(x, b2_w, b2_b, b2_g, b2_beta, b2_sw, b2_sb, b2_sg, b2_sbeta)
    new_points = _max_over_samples(x)
    return (new_xyz, new_points)


# speedup vs baseline: 3.4867x; 2.1867x over previous
import functools

import jax
import jax.numpy as jnp
from jax import lax
from jax.experimental import pallas as pl
from jax.experimental.pallas import tpu as pltpu
from jax.experimental.pallas import tpu_sc as plsc

NPOINT = 1024
NSAMPLE = 32
B = 8
N = 4096
CT = 128
ONE_COL = 67
NPAIR = B * NPOINT * NSAMPLE
NQ = B * NPOINT


def _fps_pallas_kernel(p_ref, c_ref):
    _, b, n = p_ref.shape
    x = p_ref[0]
    y = p_ref[1]
    z = p_ref[2]
    iota = jax.lax.broadcasted_iota(jnp.int32, (b, n), 1)
    iota_p = jax.lax.broadcasted_iota(jnp.int32, (b, NPOINT), 1)

    def body(i, carry):
        dists, far, cx, cy, cz = carry
        onehot = iota == far
        fx = jnp.sum(jnp.where(onehot, x, 0.0), axis=1, keepdims=True)
        fy = jnp.sum(jnp.where(onehot, y, 0.0), axis=1, keepdims=True)
        fz = jnp.sum(jnp.where(onehot, z, 0.0), axis=1, keepdims=True)
        slot = iota_p == i
        cx = jnp.where(slot, fx, cx)
        cy = jnp.where(slot, fy, cy)
        cz = jnp.where(slot, fz, cz)
        d = (x - fx) ** 2 + (y - fy) ** 2 + (z - fz) ** 2
        dists = jnp.minimum(dists, d)
        m = jnp.max(dists, axis=1, keepdims=True)
        far = jnp.min(jnp.where(dists == m, iota, n), axis=1, keepdims=True)
        return dists, far, cx, cy, cz

    dists0 = jnp.full((b, n), jnp.inf, dtype=jnp.float32)
    far0 = jnp.zeros((b, 1), dtype=jnp.int32)
    c0 = jnp.zeros((b, NPOINT), dtype=jnp.float32)
    _, _, cx, cy, cz = jax.lax.fori_loop(
        0, NPOINT, body, (dists0, far0, c0, c0, c0))
    c_ref[0] = cx
    c_ref[1] = cy
    c_ref[2] = cz


def _fps_new_xyz(xyz, interpret=False):
    b, n, _ = xyz.shape
    planes = jnp.transpose(xyz, (2, 0, 1))
    c = pl.pallas_call(
        _fps_pallas_kernel,
        out_shape=jax.ShapeDtypeStruct((3, b, NPOINT), jnp.float32),
        interpret=interpret,
    )(planes)
    return jnp.transpose(c, (1, 2, 0))


def _square_distance(src, dst):
    dist = -2.0 * jnp.matmul(src, jnp.transpose(dst, (0, 2, 1)))
    dist = dist + jnp.sum(src ** 2, axis=-1)[:, :, None]
    dist = dist + jnp.sum(dst ** 2, axis=-1)[:, None, :]
    return dist


def _knn_idx(xyz, new_xyz):
    sqrdists = _square_distance(new_xyz, xyz)
    _, idx = jax.lax.top_k(-sqrdists, NSAMPLE)
    return idx


def _sc_gather(tab, gidx2d):
    mesh = plsc.VectorSubcoreMesh(core_axis_name="c", subcore_axis_name="s")

    @functools.partial(
        pl.kernel, mesh=mesh,
        out_type=jax.ShapeDtypeStruct((NPAIR, CT), jnp.float32),
        scratch_types=[
            pltpu.VMEM((128,), jnp.int32),
            pltpu.VMEM((128, CT), jnp.float32),
            pltpu.SemaphoreType.DMA,
        ],
    )
    def k(tab_hbm, idx_hbm, out_hbm, idx_v, rows_v, sem):
        wid = lax.axis_index("s") * 2 + lax.axis_index("c")

        def body(i, _):
            row = wid * 64 + i
            pltpu.sync_copy(idx_hbm.at[row], idx_v)
            pltpu.async_copy(tab_hbm.at[idx_v], rows_v, sem).wait()
            pltpu.sync_copy(rows_v, out_hbm.at[pl.ds(row * 128, 128)])
            return 0

        jax.lax.fori_loop(0, 64, body, 0, unroll=False)

    return k(tab, gidx2d)


def _center(g_blk, nx_blk):
    s, q, _ = g_blk.shape
    x = g_blk.reshape(s * q, CT)
    r0 = jax.lax.broadcasted_iota(jnp.int32, (s * q, q), 0)
    r1 = jax.lax.broadcasted_iota(jnp.int32, (s * q, q), 1)
    rmat = (lax.rem(r0, q) == r1).astype(jnp.float32)
    qpad = jnp.dot(rmat, nx_blk, preferred_element_type=jnp.float32)
    return x - qpad


def _m1_kernel(g_ref, nx_ref, gram_ref):
    xc = _center(g_ref[...], nx_ref[...])
    g = lax.dot_general(xc, xc, (((0,), (0,)), ((), ())),
                        preferred_element_type=jnp.float32)
    @pl.when(pl.program_id(0) == 0)
    def _():
        gram_ref[...] = jnp.zeros_like(gram_ref)
    gram_ref[...] += g


def _m2_kernel(g_ref, nx_ref, w_ref, x2_ref, gram_ref, s_ref):
    xc = _center(g_ref[...], nx_ref[...])
    y = lax.dot_general(xc, w_ref[...], (((1,), (1,)), ((), ())),
                        preferred_element_type=jnp.float32)
    main = y[:, 0:64]
    short = y[:, 64:128]
    x2 = main * jax.nn.sigmoid(main) + short
    s, q, _ = g_ref.shape
    x2_ref[...] = x2.reshape(s, q, 64)
    g2 = lax.dot_general(x2, x2, (((0,), (0,)), ((), ())),
                         preferred_element_type=jnp.float32)
    s2 = jnp.broadcast_to(jnp.sum(x2, axis=0, keepdims=True), (8, 64))
    @pl.when(pl.program_id(0) == 0)
    def _():
        gram_ref[...] = jnp.zeros_like(gram_ref)
        s_ref[...] = jnp.zeros_like(s_ref)
    gram_ref[...] += g2
    s_ref[...] += s2


def _m3_kernel(x2_ref, w_ref, b_ref, o_ref):
    s, q, _ = x2_ref.shape
    x2 = x2_ref[...].reshape(s * q, 64)
    yt = lax.dot_general(w_ref[...], x2, (((1,), (1,)), ((), ())),
                         preferred_element_type=jnp.float32)
    yt = yt + b_ref[:, 0:1]
    main = yt[0:128]
    short = yt[128:256]
    z = main * jax.nn.sigmoid(main) + short
    o_ref[...] = jnp.max(z.reshape(128, s, q), axis=1)


def _fold_affine(mean_vec, cov, w, bias, gamma, beta, eps=1e-5):
    mean_y = w @ mean_vec + bias
    var_y = jnp.einsum('oc,cd,od->o', w, cov, w)
    s = gamma / jnp.sqrt(var_y + eps)
    return s[:, None] * w, beta + s * (bias - mean_y)


def _mlp(grouped, new_xyz, params, interpret=False):
    (b1_w, b1_b, b1_g, b1_beta, b1_sw, b1_sb, b1_sg, b1_sbeta,
     b2_w, b2_b, b2_g, b2_beta, b2_sw, b2_sb, b2_sg, b2_sbeta) = params
    g3 = grouped.reshape(NSAMPLE, NQ, CT)
    nxpad = jnp.pad(new_xyz.reshape(NQ, 3), ((0, 0), (0, CT - 3)))

    nblk1 = 32
    qb1 = NQ // nblk1
    gram1 = pl.pallas_call(
        _m1_kernel,
        grid=(nblk1,),
        in_specs=[
            pl.BlockSpec((NSAMPLE, qb1, CT), lambda i: (0, i, 0)),
            pl.BlockSpec((qb1, CT), lambda i: (i, 0)),
        ],
        out_specs=pl.BlockSpec((CT, CT), lambda i: (0, 0)),
        out_shape=jax.ShapeDtypeStruct((CT, CT), jnp.float32),
        interpret=interpret,
    )(g3, nxpad)

    m = jnp.float32(NPAIR)
    mean1 = gram1[ONE_COL] / m
    cov1 = gram1 / m - jnp.outer(mean1, mean1)
    w1e = jnp.zeros((64, CT), jnp.float32).at[:, 0:ONE_COL].set(b1_w)
    w1se = jnp.zeros((64, CT), jnp.float32).at[:, 0:ONE_COL].set(b1_sw)
    a1, c1 = _fold_affine(mean1, cov1, w1e, b1_b, b1_g, b1_beta)
    a1s, c1s = _fold_affine(mean1, cov1, w1se, b1_sb, b1_sg, b1_sbeta)
    w1cat = jnp.concatenate([a1, a1s], axis=0)
    w1cat = w1cat.at[:, ONE_COL].add(jnp.concatenate([c1, c1s]))

    x2, gram2, s2 = pl.pallas_call(
        _m2_kernel,
        grid=(nblk1,),
        in_specs=[
            pl.BlockSpec((NSAMPLE, qb1, CT), lambda i: (0, i, 0)),
            pl.BlockSpec((qb1, CT), lambda i: (i, 0)),
            pl.BlockSpec((128, CT), lambda i: (0, 0)),
        ],
        out_specs=[
            pl.BlockSpec((NSAMPLE, qb1, 64), lambda i: (0, i, 0)),
            pl.BlockSpec((64, 64), lambda i: (0, 0)),
            pl.BlockSpec((8, 64), lambda i: (0, 0)),
        ],
        out_shape=[
            jax.ShapeDtypeStruct((NSAMPLE, NQ, 64), jnp.float32),
            jax.ShapeDtypeStruct((64, 64), jnp.float32),
            jax.ShapeDtypeStruct((8, 64), jnp.float32),
        ],
        interpret=interpret,
    )(g3, nxpad, w1cat)

    mean2 = s2[0] / m
    cov2 = gram2 / m - jnp.outer(mean2, mean2)
    a2, c2 = _fold_affine(mean2, cov2, b2_w, b2_b, b2_g, b2_beta)
    a2s, c2s = _fold_affine(mean2, cov2, b2_sw, b2_sb, b2_sg, b2_sbeta)
    w2cat = jnp.concatenate([a2, a2s], axis=0)
    b2cat = jnp.concatenate([c2, c2s])
    b2pad = jnp.broadcast_to(b2cat[:, None], (256, 8))

    nblk3 = 64
    qb3 = NQ // nblk3
    out = pl.pallas_call(
        _m3_kernel,
        grid=(nblk3,),
        in_specs=[
            pl.BlockSpec((NSAMPLE, qb3, 64), lambda i: (0, i, 0)),
            pl.BlockSpec((256, 64), lambda i: (0, 0)),
            pl.BlockSpec((256, 8), lambda i: (0, 0)),
        ],
        out_specs=pl.BlockSpec((128, qb3), lambda i: (0, i)),
        out_shape=jax.ShapeDtypeStruct((128, NQ), jnp.float32),
        interpret=interpret,
    )(x2, w2cat, b2pad)
    return out.reshape(128, B, NPOINT).transpose(1, 0, 2)


def kernel(xyz, points, b1_w, b1_b, b1_g, b1_beta, b1_sw, b1_sb, b1_sg, b1_sbeta, b2_w, b2_b, b2_g, b2_beta, b2_sw, b2_sb, b2_sg, b2_sbeta):
    new_xyz = _fps_new_xyz(xyz)
    idx = _knn_idx(xyz, new_xyz)
    ones = jnp.ones((B, N, 1), jnp.float32)
    tab = jnp.concatenate([xyz, points, ones], axis=-1)
    tab = jnp.pad(tab, ((0, 0), (0, 0), (0, CT - 68))).reshape(B * N, CT)
    gidx = (idx + (jnp.arange(B, dtype=jnp.int32) * N)[:, None, None])
    gidx = gidx.transpose(2, 0, 1).reshape(NPAIR // 128, 128).astype(jnp.int32)
    grouped = _sc_gather(tab, gidx)
    params = (b1_w, b1_b, b1_g, b1_beta, b1_sw, b1_sb, b1_sg, b1_sbeta,
              b2_w, b2_b, b2_g, b2_beta, b2_sw, b2_sb, b2_sg, b2_sbeta)
    new_points = _mlp(grouped, new_xyz, params)
    return (new_xyz, new_points)
